# baseline (speedup 1.0000x reference)
import functools

import jax
import jax.numpy as jnp
from jax import lax
from jax.experimental import pallas as pl
from jax.experimental.pallas import tpu as pltpu
from jax.experimental.pallas import tpu_sc as plsc

B = 8
NCH = 96
H = W = 384
NC = 1
NS = 16
NW = NC * NS
PW = NW // B
ROWS = H // PW
HALF = ROWS // 2


def _body(in_hbm, lab_hbm, out_hbm, lab_v, buf0, buf1, s0, s1, s2, s3):
    c = lax.axis_index("c")
    s = lax.axis_index("s")
    wid = s * NC + c
    b = wid // PW
    part = wid % PW

    pltpu.sync_copy(lab_hbm, lab_v.at[pl.ds(0, B)])
    lv = lab_v[pl.ds(b, 16)]
    label_b = lv[0]

    r0 = part * ROWS
    i0 = pltpu.async_copy(in_hbm.at[b, label_b, pl.ds(r0, HALF)], buf0, s0)
    i1 = pltpu.async_copy(in_hbm.at[b, label_b, pl.ds(r0 + HALF, HALF)], buf1, s1)
    i0.wait()
    o0 = pltpu.async_copy(buf0, out_hbm.at[b, pl.ds(r0, HALF)], s2)
    i1.wait()
    o1 = pltpu.async_copy(buf1, out_hbm.at[b, pl.ds(r0 + HALF, HALF)], s3)
    o0.wait()
    o1.wait()


def kernel(in_feat_map, labels):
    mesh = plsc.VectorSubcoreMesh(core_axis_name="c", subcore_axis_name="s", num_cores=1)
    run = functools.partial(
        pl.kernel,
        mesh=mesh,
        out_type=jax.ShapeDtypeStruct((B, H, W), jnp.float32),
        scratch_types=[
            pltpu.VMEM((32,), jnp.int32),
            pltpu.VMEM((HALF, W), jnp.float32),
            pltpu.VMEM((HALF, W), jnp.float32),
            pltpu.SemaphoreType.DMA,
            pltpu.SemaphoreType.DMA,
            pltpu.SemaphoreType.DMA,
            pltpu.SemaphoreType.DMA,
        ],
    )(_body)
    return run(in_feat_map, labels.astype(jnp.int32))


# speedup vs baseline: 1.0544x; 1.0544x over previous
import functools

import jax
import jax.numpy as jnp
from jax import lax
from jax.experimental import pallas as pl
from jax.experimental.pallas import tpu as pltpu
from jax.experimental.pallas import tpu_sc as plsc

B = 8
NCH = 96
H = W = 384
NC = 2
NS = 16
NW = NC * NS
PW = NW // B
ROWS = H // PW


def _body(in_hbm, lab_hbm, out_hbm, lab_v, buf):
    c = lax.axis_index("c")
    s = lax.axis_index("s")
    wid = s * NC + c
    b = wid // PW
    part = wid % PW

    pltpu.sync_copy(lab_hbm, lab_v.at[pl.ds(0, B)])
    lv = lab_v[pl.ds(b, 16)]
    label_b = lv[0]

    r0 = part * ROWS
    pltpu.sync_copy(in_hbm.at[b, label_b, pl.ds(r0, ROWS)], buf)
    pltpu.sync_copy(buf, out_hbm.at[b, pl.ds(r0, ROWS)])


def kernel(in_feat_map, labels):
    mesh = plsc.VectorSubcoreMesh(core_axis_name="c", subcore_axis_name="s")
    run = functools.partial(
        pl.kernel,
        mesh=mesh,
        out_type=jax.ShapeDtypeStruct((B, H, W), jnp.float32),
        scratch_types=[
            pltpu.VMEM((32,), jnp.int32),
            pltpu.VMEM((ROWS, W), jnp.float32),
        ],
    )(_body)
    return run(in_feat_map, labels.astype(jnp.int32))
